# baseline (speedup 1.0000x reference)
import functools

import jax
import jax.numpy as jnp
from jax import lax
from jax.experimental import pallas as pl
from jax.experimental.pallas import tpu as pltpu
from jax.experimental.pallas import tpu_sc as plsc

_NUM_CORES = 2
_NUM_SUBCORES = 16
_LANES = 16
_CHUNK = 128
_NBUF = 6
_GDEPTH = 3


@jax.jit
def _selection_gather(x, idx):
    n_batches, _, d = x.shape
    k = idx.shape[1]
    n_chunks = k // _CHUNK
    mesh = plsc.VectorSubcoreMesh(core_axis_name="c", subcore_axis_name="s")

    @functools.partial(
        pl.kernel,
        out_type=jax.ShapeDtypeStruct((n_batches, k, d), jnp.float32),
        mesh=mesh,
        scratch_types=[
            pltpu.VMEM((k,), jnp.int32),
            pltpu.VMEM((_NBUF, _CHUNK, d), jnp.float32),
            [pltpu.SemaphoreType.DMA] * _NBUF,
            [pltpu.SemaphoreType.DMA] * _NBUF,
        ],
    )
    def run(x_hbm, idx_hbm, out_hbm, idx_v, rows_v, gsem, wsem):
        b = lax.axis_index("s") * _NUM_CORES + lax.axis_index("c")

        pltpu.sync_copy(idx_hbm.at[b], idx_v)

        def start_gather(c):
            h = pltpu.make_async_copy(
                x_hbm.at[b].at[idx_v.at[pl.ds(c * _CHUNK, _CHUNK)]],
                rows_v.at[c % _NBUF],
                gsem[c % _NBUF],
            )
            h.start()
            return h

        def start_write(c):
            h = pltpu.make_async_copy(
                rows_v.at[c % _NBUF],
                out_hbm.at[b, pl.ds(c * _CHUNK, _CHUNK)],
                wsem[c % _NBUF],
            )
            h.start()
            return h

        gathers, writes = {}, {}
        for c in range(n_chunks):
            if c >= _NBUF:
                writes[c - _NBUF].wait()
            gathers[c] = start_gather(c)
            if c >= _GDEPTH:
                cc = c - _GDEPTH
                gathers[cc].wait()
                writes[cc] = start_write(cc)
        for cc in range(n_chunks - _GDEPTH, n_chunks):
            gathers[cc].wait()
            writes[cc] = start_write(cc)
        for cc in range(n_chunks - _NBUF, n_chunks):
            writes[cc].wait()

    return run(x, idx)


def kernel(x, index):
    return _selection_gather(x, index.astype(jnp.int32))
